# baseline (speedup 1.0000x reference)
import functools

import jax
import jax.numpy as jnp
from jax.experimental import pallas as pl
from jax.experimental.pallas import tpu as pltpu


def _lsh_kernel(x_ref, rt_ref, o_ref, *, rounds, L):
    x = x_ref[0]
    rt = rt_ref[0]
    hT = jax.lax.dot_general(
        rt, x, (((1,), (1,)), ((), ())),
        preferred_element_type=jnp.float32)
    tokens = x.shape[0]
    iota = jax.lax.broadcasted_iota(jnp.int32, (L, tokens), 0)
    parts = []
    for k in range(rounds):
        hk = hT[k * L:(k + 1) * L, :]
        c = jnp.abs(hk)
        w = iota + jnp.where(hk < 0, L, 0)
        m = jnp.max(c, axis=0, keepdims=True)
        idx = jnp.where(c == m, w, 2 * L)
        parts.append(jnp.min(idx, axis=0, keepdims=True))
    o_ref[0] = jnp.concatenate(parts, axis=0)


def kernel(inp, rand_matrix, n_buckets):
    B, S, D = inp.shape
    _, _, R, L = rand_matrix.shape
    rt = rand_matrix.transpose(0, 2, 3, 1).reshape(B, R * L, D)
    CHUNK = 1024
    grid = (B, S // CHUNK)
    out = pl.pallas_call(
        functools.partial(_lsh_kernel, rounds=R, L=L),
        grid=grid,
        in_specs=[
            pl.BlockSpec((1, CHUNK, D), lambda b, s: (b, s, 0)),
            pl.BlockSpec((1, R * L, D), lambda b, s: (b, 0, 0)),
        ],
        out_specs=pl.BlockSpec((1, R, CHUNK), lambda b, s: (b, 0, s)),
        out_shape=jax.ShapeDtypeStruct((B, R, S), jnp.int32),
        compiler_params=pltpu.CompilerParams(
            dimension_semantics=("parallel", "parallel"),
        ),
    )(inp, rt)
    return out.transpose(0, 2, 1)


# speedup vs baseline: 1.4730x; 1.4730x over previous
import functools

import jax
import jax.numpy as jnp
from jax.experimental import pallas as pl
from jax.experimental.pallas import tpu as pltpu


def _lsh_kernel(x_ref, rt_ref, o_ref, *, rounds, L):
    x = x_ref[0]
    rt = rt_ref[0]
    hT = jax.lax.dot_general(
        rt, x, (((1,), (1,)), ((), ())),
        preferred_element_type=jnp.float32)
    tokens = x.shape[0]
    iota = jax.lax.broadcasted_iota(
        jnp.int32, (L, tokens), 0).astype(jnp.float32)
    parts = []
    for k in range(rounds):
        hk = hT[k * L:(k + 1) * L, :]
        c = jnp.abs(hk)
        w = iota + jnp.where(hk < 0, float(L), 0.0)
        m = jnp.max(c, axis=0, keepdims=True)
        idx = jnp.where(c == m, w, float(2 * L))
        parts.append(jnp.min(idx, axis=0, keepdims=True).astype(jnp.int32))
    o_ref[0] = jnp.concatenate(parts, axis=0)


def kernel(inp, rand_matrix, n_buckets):
    B, S, D = inp.shape
    _, _, R, L = rand_matrix.shape
    rt = rand_matrix.transpose(0, 2, 3, 1).reshape(B, R * L, D)
    CHUNK = 2048
    grid = (B, S // CHUNK)
    out = pl.pallas_call(
        functools.partial(_lsh_kernel, rounds=R, L=L),
        grid=grid,
        in_specs=[
            pl.BlockSpec((1, CHUNK, D), lambda b, s: (b, s, 0)),
            pl.BlockSpec((1, R * L, D), lambda b, s: (b, 0, 0)),
        ],
        out_specs=pl.BlockSpec((1, R, CHUNK), lambda b, s: (b, 0, s)),
        out_shape=jax.ShapeDtypeStruct((B, R, S), jnp.int32),
        compiler_params=pltpu.CompilerParams(
            dimension_semantics=("parallel", "parallel"),
        ),
    )(inp, rt)
    return out.transpose(0, 2, 1)


# speedup vs baseline: 1.4767x; 1.0026x over previous
import functools

import jax
import jax.numpy as jnp
from jax.experimental import pallas as pl
from jax.experimental.pallas import tpu as pltpu


def _lsh_kernel(x_ref, rt_ref, o_ref, *, rounds, L):
    x = x_ref[0]
    rt = rt_ref[0]
    hT = jax.lax.dot_general(
        rt, x, (((1,), (1,)), ((), ())),
        preferred_element_type=jnp.float32)
    tokens = x.shape[0]
    iota = jax.lax.broadcasted_iota(
        jnp.int32, (L, tokens), 0).astype(jnp.float32)
    iota_neg = iota + float(L)
    parts = []
    for k in range(rounds):
        hk = hT[k * L:(k + 1) * L, :]
        c = jnp.abs(hk)
        w = jnp.where(hk < 0, iota_neg, iota)
        m = jnp.max(c, axis=0, keepdims=True)
        idx = jnp.where(c == m, w, float(2 * L))
        parts.append(jnp.min(idx, axis=0, keepdims=True).astype(jnp.int32))
    o_ref[0] = jnp.concatenate(parts, axis=0)


def kernel(inp, rand_matrix, n_buckets):
    B, S, D = inp.shape
    _, _, R, L = rand_matrix.shape
    rt = rand_matrix.transpose(0, 2, 3, 1).reshape(B, R * L, D)
    CHUNK = 4096
    grid = (B, S // CHUNK)
    out = pl.pallas_call(
        functools.partial(_lsh_kernel, rounds=R, L=L),
        grid=grid,
        in_specs=[
            pl.BlockSpec((1, CHUNK, D), lambda b, s: (b, s, 0)),
            pl.BlockSpec((1, R * L, D), lambda b, s: (b, 0, 0)),
        ],
        out_specs=pl.BlockSpec((1, R, CHUNK), lambda b, s: (b, 0, s)),
        out_shape=jax.ShapeDtypeStruct((B, R, S), jnp.int32),
        compiler_params=pltpu.CompilerParams(
            dimension_semantics=("parallel", "parallel"),
        ),
    )(inp, rt)
    return out.transpose(0, 2, 1)
